# Initial kernel scaffold; baseline (speedup 1.0000x reference)
import jax
import jax.numpy as jnp
from jax.experimental import pallas as pl


def kernel(X, means, chols, weights, it):
    raise NotImplementedError("write your pallas kernel here")


# speedup vs baseline: 10.2608x; 10.2608x over previous
import functools
import math

import jax
import jax.numpy as jnp
from jax.experimental import pallas as pl
from jax.experimental.pallas import tpu as pltpu

_K = 32
_C = 512
_BR = 64
_P = 2
_NEG_HALF_LOG2E = -0.5 * math.log2(math.e)
_INV_TWO_PI = 1.0 / (2.0 * math.pi)


def _cluster_rows(pr):
    c00 = pr[0:1, :]
    c10 = pr[1:2, :]
    c11 = pr[2:3, :]
    m0 = pr[3:4, :]
    m1 = pr[4:5, :]
    w = pr[5:6, :]
    l00sq = c00 * c00 + 1.0
    l00 = jnp.sqrt(l00sq)
    l10 = c00 * c10 / l00
    l11sq = c10 * c10 + c11 * c11 + 1.0 - l10 * l10
    l11 = jnp.sqrt(l11sq)
    a = 1.0 / l00
    cc = 1.0 / l11
    b = -(l10 * a * cc)
    coef = w * _INV_TWO_PI * jax.lax.rsqrt(l00 * l11)
    h = _NEG_HALF_LOG2E
    am = h * a
    bm = h * b
    cm = h * cc
    dm = h * (-(2.0 * a * m0 + b * m1))
    em = h * (-(2.0 * cc * m1 + b * m0))
    fm = h * (a * m0 * m0 + b * m0 * m1 + cc * m1 * m1)
    coef2 = coef * jnp.exp2(fm)
    return am, bm, cm, dm, em, coef2


def _cluster_cols(pc):
    c00 = pc[:, 0:1]
    c10 = pc[:, 1:2]
    c11 = pc[:, 2:3]
    m0 = pc[:, 3:4]
    m1 = pc[:, 4:5]
    w = pc[:, 5:6]
    l00 = jnp.sqrt(c00 * c00 + 1.0)
    l10 = c00 * c10 / l00
    l11 = jnp.sqrt(c10 * c10 + c11 * c11 + 1.0 - l10 * l10)
    return l00, l10, l11, m0, m1, w


def _cluster_rows_l(pr):
    c00 = pr[0:1, :]
    c10 = pr[1:2, :]
    c11 = pr[2:3, :]
    l00 = jnp.sqrt(c00 * c00 + 1.0)
    l10 = c00 * c10 / l00
    l11 = jnp.sqrt(c10 * c10 + c11 * c11 + 1.0 - l10 * l10)
    return l00, l10, l11, pr[3:4, :], pr[4:5, :], pr[5:6, :]


def kernel(X, means, chols, weights, it):
    del it
    n = X.shape[0]
    s_steps = -(-n // (_P * _BR * _C))
    r_rows = _P * s_steps * _BR
    pad = r_rows * _C - n

    x0 = jnp.pad(X[:, 0], (0, pad)).reshape(r_rows, _C)
    x1 = jnp.pad(X[:, 1], (0, pad)).reshape(r_rows, _C)
    pr = jnp.stack(
        [chols[:, 0, 0], chols[:, 1, 0], chols[:, 1, 1],
         means[:, 0], means[:, 1], weights]
    ).astype(jnp.float32)
    pc = pr.T

    def body(x0_ref, x1_ref, pr_ref, pc_ref, out_ref, z_ref, acc_ref, tbl_ref):
        j = pl.program_id(1)

        @pl.when(j == 0)
        def _prep():
            prv = pr_ref[...]
            rows = _cluster_rows(prv)
            for i, row in enumerate(rows):
                for k in range(_K):
                    tbl_ref[i, k] = row[0, k]
            l00c, l10c, l11c, m0c, m1c, wc = _cluster_cols(pc_ref[...])
            l00r, l10r, l11r, m0r, m1r, wr = _cluster_rows_l(prv)
            m00 = l00c + l00r
            m10 = l10c + l10r
            m11 = l11c + l11r
            dmu0 = m0c - m0r
            dmu1 = m1c - m1r
            r00 = 1.0 / m00
            r11 = 1.0 / m11
            qz = dmu0 * dmu0 * r00 - m10 * r00 * r11 * dmu0 * dmu1 \
                + dmu1 * dmu1 * r11
            zt = jnp.exp2(_NEG_HALF_LOG2E * qz) * _INV_TWO_PI \
                * jax.lax.rsqrt(m00 * m11)
            z_ref[...] = jnp.sum(zt * (wc * wr)).reshape(1, 1, 1)

        x0b = x0_ref[...]
        x1b = x1_ref[...]
        p0 = x0b * x0b
        p1 = x0b * x1b
        p2 = x1b * x1b
        s = None
        for k in range(_K):
            ak = tbl_ref[0, k]
            bk = tbl_ref[1, k]
            ck = tbl_ref[2, k]
            dk = tbl_ref[3, k]
            ek = tbl_ref[4, k]
            cfk = tbl_ref[5, k]
            g = p0 * ak + p1 * bk + p2 * ck + x0b * dk + x1b * ek
            t = cfk * jnp.exp2(g)
            s = t if s is None else s + t

        row0 = (pl.program_id(0) * s_steps + j) * _BR
        ir = jax.lax.broadcasted_iota(jnp.int32, (_BR, _C), 0)
        ic = jax.lax.broadcasted_iota(jnp.int32, (_BR, _C), 1)
        idx = (row0 + ir) * _C + ic
        s = jnp.where(idx < n, s, 0.0)
        s2 = s * s
        part = s2[0:8]
        for i in range(1, _BR // 8):
            part = part + s2[8 * i:8 * i + 8]

        @pl.when(j == 0)
        def _init():
            acc_ref[...] = part

        @pl.when(j > 0)
        def _acc():
            acc_ref[...] += part

        @pl.when(j == s_steps - 1)
        def _flush():
            out_ref[...] = jnp.sum(acc_ref[...]).reshape(1, 1, 1)

    partials, zval = pl.pallas_call(
        body,
        grid=(_P, s_steps),
        in_specs=[
            pl.BlockSpec((_BR, _C), lambda p, j: (p * s_steps + j, 0)),
            pl.BlockSpec((_BR, _C), lambda p, j: (p * s_steps + j, 0)),
            pl.BlockSpec((6, _K), lambda p, j: (0, 0)),
            pl.BlockSpec((_K, 6), lambda p, j: (0, 0)),
        ],
        out_specs=[
            pl.BlockSpec((1, 1, 1), lambda p, j: (p, 0, 0)),
            pl.BlockSpec((1, 1, 1), lambda p, j: (p, 0, 0)),
        ],
        out_shape=[
            jax.ShapeDtypeStruct((_P, 1, 1), jnp.float32),
            jax.ShapeDtypeStruct((_P, 1, 1), jnp.float32),
        ],
        scratch_shapes=[
            pltpu.VMEM((8, _C), jnp.float32),
            pltpu.SMEM((6, _K), jnp.float32),
        ],
        compiler_params=pltpu.CompilerParams(
            dimension_semantics=("parallel", "arbitrary"),
        ),
        name="nmsq_gm_nll",
    )(x0, x1, pr, pc)

    total = jnp.sum(partials)
    return -(jnp.log(total) - jnp.log(zval[0, 0, 0])) / n


# speedup vs baseline: 11.2196x; 1.0934x over previous
import functools
import math

import jax
import jax.numpy as jnp
from jax.experimental import pallas as pl
from jax.experimental.pallas import tpu as pltpu

_K = 32
_C = 512
_BR = 64
_SB = 8
_P = 1
_NEG_HALF_LOG2E = -0.5 * math.log2(math.e)
_INV_TWO_PI = 1.0 / (2.0 * math.pi)


def _cluster_rows(pr):
    c00 = pr[0:1, :]
    c10 = pr[1:2, :]
    c11 = pr[2:3, :]
    m0 = pr[3:4, :]
    m1 = pr[4:5, :]
    w = pr[5:6, :]
    l00sq = c00 * c00 + 1.0
    l00 = jnp.sqrt(l00sq)
    l10 = c00 * c10 / l00
    l11sq = c10 * c10 + c11 * c11 + 1.0 - l10 * l10
    l11 = jnp.sqrt(l11sq)
    a = 1.0 / l00
    cc = 1.0 / l11
    b = -(l10 * a * cc)
    coef = w * _INV_TWO_PI * jax.lax.rsqrt(l00 * l11)
    h = _NEG_HALF_LOG2E
    am = h * a
    bm = h * b
    cm = h * cc
    dm = h * (-(2.0 * a * m0 + b * m1))
    em = h * (-(2.0 * cc * m1 + b * m0))
    fm = h * (a * m0 * m0 + b * m0 * m1 + cc * m1 * m1)
    coef2 = coef * jnp.exp2(fm)
    return am, bm, cm, dm, em, coef2


def _cluster_cols(pc):
    c00 = pc[:, 0:1]
    c10 = pc[:, 1:2]
    c11 = pc[:, 2:3]
    m0 = pc[:, 3:4]
    m1 = pc[:, 4:5]
    w = pc[:, 5:6]
    l00 = jnp.sqrt(c00 * c00 + 1.0)
    l10 = c00 * c10 / l00
    l11 = jnp.sqrt(c10 * c10 + c11 * c11 + 1.0 - l10 * l10)
    return l00, l10, l11, m0, m1, w


def _cluster_rows_l(pr):
    c00 = pr[0:1, :]
    c10 = pr[1:2, :]
    c11 = pr[2:3, :]
    l00 = jnp.sqrt(c00 * c00 + 1.0)
    l10 = c00 * c10 / l00
    l11 = jnp.sqrt(c10 * c10 + c11 * c11 + 1.0 - l10 * l10)
    return l00, l10, l11, pr[3:4, :], pr[4:5, :], pr[5:6, :]


def kernel(X, means, chols, weights, it):
    del it
    n = X.shape[0]
    s_steps = -(-n // (_P * _BR * _C))
    r_rows = _P * s_steps * _BR
    pad = r_rows * _C - n

    xt = jnp.pad(X, ((0, pad), (0, 0))).T
    x0 = xt[0].reshape(r_rows, _C)
    x1 = xt[1].reshape(r_rows, _C)
    pr = jnp.stack(
        [chols[:, 0, 0], chols[:, 1, 0], chols[:, 1, 1],
         means[:, 0], means[:, 1], weights]
    ).astype(jnp.float32)
    pc = pr.T

    def body(x0_ref, x1_ref, pr_ref, pc_ref, out_ref, z_ref, acc_ref, tbl_ref):
        j = pl.program_id(1)

        @pl.when(j == 0)
        def _prep():
            prv = pr_ref[...]
            rows = _cluster_rows(prv)
            for i, row in enumerate(rows):
                for k in range(_K):
                    tbl_ref[i, k] = row[0, k]
            l00c, l10c, l11c, m0c, m1c, wc = _cluster_cols(pc_ref[...])
            l00r, l10r, l11r, m0r, m1r, wr = _cluster_rows_l(prv)
            m00 = l00c + l00r
            m10 = l10c + l10r
            m11 = l11c + l11r
            dmu0 = m0c - m0r
            dmu1 = m1c - m1r
            r00 = 1.0 / m00
            r11 = 1.0 / m11
            qz = dmu0 * dmu0 * r00 - m10 * r00 * r11 * dmu0 * dmu1 \
                + dmu1 * dmu1 * r11
            zt = jnp.exp2(_NEG_HALF_LOG2E * qz) * _INV_TWO_PI \
                * jax.lax.rsqrt(m00 * m11)
            z_ref[...] = jnp.sum(zt * (wc * wr)).reshape(1, 1, 1)

        sc = [[tbl_ref[i, k] for i in range(6)] for k in range(_K)]

        row0 = (pl.program_id(0) * s_steps + j) * _BR
        ir = jax.lax.broadcasted_iota(jnp.int32, (_SB, _C), 0)
        ic = jax.lax.broadcasted_iota(jnp.int32, (_SB, _C), 1)

        acc = None
        for rb in range(0, _BR, _SB):
            x0s = x0_ref[rb:rb + _SB, :]
            x1s = x1_ref[rb:rb + _SB, :]
            p0 = x0s * x0s
            p1 = x0s * x1s
            p2 = x1s * x1s
            s = None
            for k in range(_K):
                ak, bk, ck, dk, ek, cfk = sc[k]
                g = p0 * ak + p1 * bk + p2 * ck + x0s * dk + x1s * ek
                t = cfk * jnp.exp2(g)
                s = t if s is None else s + t
            idx = (row0 + rb + ir) * _C + ic
            s = jnp.where(idx < n, s, 0.0)
            t2 = s * s
            acc = t2 if acc is None else acc + t2

        @pl.when(j == 0)
        def _init():
            acc_ref[...] = acc

        @pl.when(j > 0)
        def _acc():
            acc_ref[...] += acc

        @pl.when(j == s_steps - 1)
        def _flush():
            out_ref[...] = jnp.sum(acc_ref[...]).reshape(1, 1, 1)

    partials, zval = pl.pallas_call(
        body,
        grid=(_P, s_steps),
        in_specs=[
            pl.BlockSpec((_BR, _C), lambda p, j: (p * s_steps + j, 0)),
            pl.BlockSpec((_BR, _C), lambda p, j: (p * s_steps + j, 0)),
            pl.BlockSpec((6, _K), lambda p, j: (0, 0)),
            pl.BlockSpec((_K, 6), lambda p, j: (0, 0)),
        ],
        out_specs=[
            pl.BlockSpec((1, 1, 1), lambda p, j: (p, 0, 0)),
            pl.BlockSpec((1, 1, 1), lambda p, j: (p, 0, 0)),
        ],
        out_shape=[
            jax.ShapeDtypeStruct((_P, 1, 1), jnp.float32),
            jax.ShapeDtypeStruct((_P, 1, 1), jnp.float32),
        ],
        scratch_shapes=[
            pltpu.VMEM((_SB, _C), jnp.float32),
            pltpu.SMEM((6, _K), jnp.float32),
        ],
        compiler_params=pltpu.CompilerParams(
            dimension_semantics=("parallel", "arbitrary"),
        ),
        name="nmsq_gm_nll",
    )(x0, x1, pr, pc)

    total = jnp.sum(partials)
    return -(jnp.log(total) - jnp.log(zval[0, 0, 0])) / n


# speedup vs baseline: 12.8823x; 1.1482x over previous
import functools
import math

import jax
import jax.numpy as jnp
from jax.experimental import pallas as pl
from jax.experimental.pallas import tpu as pltpu

_K = 32
_C = 512
_BR = 64
_SB = 8
_P = 1
_NEG_HALF_LOG2E = -0.5 * math.log2(math.e)
_INV_TWO_PI = 1.0 / (2.0 * math.pi)


def _cluster_rows(pr):
    c00 = pr[0:1, :]
    c10 = pr[1:2, :]
    c11 = pr[2:3, :]
    m0 = pr[3:4, :]
    m1 = pr[4:5, :]
    w = pr[5:6, :]
    l00sq = c00 * c00 + 1.0
    l00 = jnp.sqrt(l00sq)
    l10 = c00 * c10 / l00
    l11sq = c10 * c10 + c11 * c11 + 1.0 - l10 * l10
    l11 = jnp.sqrt(l11sq)
    a = 1.0 / l00
    cc = 1.0 / l11
    b = -(l10 * a * cc)
    coef = w * _INV_TWO_PI * jax.lax.rsqrt(l00 * l11)
    h = _NEG_HALF_LOG2E
    am = h * a
    bm = h * b
    cm = h * cc
    dm = h * (-(2.0 * a * m0 + b * m1))
    em = h * (-(2.0 * cc * m1 + b * m0))
    fm = h * (a * m0 * m0 + b * m0 * m1 + cc * m1 * m1)
    coef2 = coef * jnp.exp2(fm)
    return am, bm, cm, dm, em, coef2


def _cluster_cols(pc):
    c00 = pc[:, 0:1]
    c10 = pc[:, 1:2]
    c11 = pc[:, 2:3]
    m0 = pc[:, 3:4]
    m1 = pc[:, 4:5]
    w = pc[:, 5:6]
    l00 = jnp.sqrt(c00 * c00 + 1.0)
    l10 = c00 * c10 / l00
    l11 = jnp.sqrt(c10 * c10 + c11 * c11 + 1.0 - l10 * l10)
    return l00, l10, l11, m0, m1, w


def _cluster_rows_l(pr):
    c00 = pr[0:1, :]
    c10 = pr[1:2, :]
    c11 = pr[2:3, :]
    l00 = jnp.sqrt(c00 * c00 + 1.0)
    l10 = c00 * c10 / l00
    l11 = jnp.sqrt(c10 * c10 + c11 * c11 + 1.0 - l10 * l10)
    return l00, l10, l11, pr[3:4, :], pr[4:5, :], pr[5:6, :]


def kernel(X, means, chols, weights, it):
    del it
    n = X.shape[0]
    s_steps = -(-n // (_P * _BR * _C))
    r_rows = _P * s_steps * _BR
    pad = r_rows * _C - n

    xp = jnp.pad(X, ((0, pad), (0, 0))).reshape(-1, 128, 2)
    xt = jnp.transpose(xp, (0, 2, 1))
    x0 = xt[:, 0, :].reshape(r_rows, _C)
    x1 = xt[:, 1, :].reshape(r_rows, _C)
    pr = jnp.stack(
        [chols[:, 0, 0], chols[:, 1, 0], chols[:, 1, 1],
         means[:, 0], means[:, 1], weights]
    ).astype(jnp.float32)
    pc = pr.T

    def body(x0_ref, x1_ref, pr_ref, pc_ref, out_ref, z_ref, acc_ref, tbl_ref):
        j = pl.program_id(1)

        @pl.when(j == 0)
        def _prep():
            prv = pr_ref[...]
            rows = _cluster_rows(prv)
            for i, row in enumerate(rows):
                for k in range(_K):
                    tbl_ref[i, k] = row[0, k]
            l00c, l10c, l11c, m0c, m1c, wc = _cluster_cols(pc_ref[...])
            l00r, l10r, l11r, m0r, m1r, wr = _cluster_rows_l(prv)
            m00 = l00c + l00r
            m10 = l10c + l10r
            m11 = l11c + l11r
            dmu0 = m0c - m0r
            dmu1 = m1c - m1r
            r00 = 1.0 / m00
            r11 = 1.0 / m11
            qz = dmu0 * dmu0 * r00 - m10 * r00 * r11 * dmu0 * dmu1 \
                + dmu1 * dmu1 * r11
            zt = jnp.exp2(_NEG_HALF_LOG2E * qz) * _INV_TWO_PI \
                * jax.lax.rsqrt(m00 * m11)
            z_ref[...] = jnp.sum(zt * (wc * wr)).reshape(1, 1, 1)

        sc = [[tbl_ref[i, k] for i in range(6)] for k in range(_K)]

        row0 = (pl.program_id(0) * s_steps + j) * _BR
        ir = jax.lax.broadcasted_iota(jnp.int32, (_SB, _C), 0)
        ic = jax.lax.broadcasted_iota(jnp.int32, (_SB, _C), 1)

        acc = None
        for rb in range(0, _BR, _SB):
            x0s = x0_ref[rb:rb + _SB, :]
            x1s = x1_ref[rb:rb + _SB, :]
            p0 = x0s * x0s
            p1 = x0s * x1s
            p2 = x1s * x1s
            s = None
            for k in range(_K):
                ak, bk, ck, dk, ek, cfk = sc[k]
                g = p0 * ak + p1 * bk + p2 * ck + x0s * dk + x1s * ek
                t = cfk * jnp.exp2(g)
                s = t if s is None else s + t
            idx = (row0 + rb + ir) * _C + ic
            s = jnp.where(idx < n, s, 0.0)
            t2 = s * s
            acc = t2 if acc is None else acc + t2

        @pl.when(j == 0)
        def _init():
            acc_ref[...] = acc

        @pl.when(j > 0)
        def _acc():
            acc_ref[...] += acc

        @pl.when(j == s_steps - 1)
        def _flush():
            out_ref[...] = jnp.sum(acc_ref[...]).reshape(1, 1, 1)

    partials, zval = pl.pallas_call(
        body,
        grid=(_P, s_steps),
        in_specs=[
            pl.BlockSpec((_BR, _C), lambda p, j: (p * s_steps + j, 0)),
            pl.BlockSpec((_BR, _C), lambda p, j: (p * s_steps + j, 0)),
            pl.BlockSpec((6, _K), lambda p, j: (0, 0)),
            pl.BlockSpec((_K, 6), lambda p, j: (0, 0)),
        ],
        out_specs=[
            pl.BlockSpec((1, 1, 1), lambda p, j: (p, 0, 0)),
            pl.BlockSpec((1, 1, 1), lambda p, j: (p, 0, 0)),
        ],
        out_shape=[
            jax.ShapeDtypeStruct((_P, 1, 1), jnp.float32),
            jax.ShapeDtypeStruct((_P, 1, 1), jnp.float32),
        ],
        scratch_shapes=[
            pltpu.VMEM((_SB, _C), jnp.float32),
            pltpu.SMEM((6, _K), jnp.float32),
        ],
        compiler_params=pltpu.CompilerParams(
            dimension_semantics=("parallel", "arbitrary"),
        ),
        name="nmsq_gm_nll",
    )(x0, x1, pr, pc)

    total = jnp.sum(partials)
    return -(jnp.log(total) - jnp.log(zval[0, 0, 0])) / n
